# baseline (speedup 1.0000x reference)
import functools

import jax
import jax.numpy as jnp
from jax import lax
from jax.experimental import pallas as pl
from jax.experimental.pallas import tpu as pltpu
from jax.experimental.pallas import tpu_sc as plsc

_B = 4096
_S = 200
_D = 64
_DP = 128
_NC = 2
_NS = 16
_NW = _NC * _NS
_NB = _B // _NW
_GA = 96
_GB = _S - _GA


def _gather_body(idx_hbm, table_hbm, out_hbm, idx_v,
                 rows0_v, rows1_v, rows2_v, rows3_v,
                 rows4_v, rows5_v, rows6_v, rows7_v,
                 sg0, sg1, sg2, sg3, sg4, sg5, sg6, sg7,
                 sw0, sw1, sw2, sw3, sw4, sw5, sw6, sw7):
    wid = lax.axis_index("s") * _NC + lax.axis_index("c")
    b0 = wid * _NB
    pltpu.sync_copy(idx_hbm.at[pl.ds(b0, _NB)], idx_v)

    def fire_gather(c, buf, sem):
        pltpu.async_copy(table_hbm.at[idx_v.at[c, pl.ds(0, _GA)]],
                         buf.at[pl.ds(0, _GA)], sem)
        pltpu.async_copy(table_hbm.at[idx_v.at[c, pl.ds(_GA, _GB)]],
                         buf.at[pl.ds(_GA, _GB)], sem)

    def wait_gather(c, buf, sem):
        pltpu.make_async_copy(table_hbm.at[idx_v.at[c, pl.ds(0, _GA)]],
                              buf.at[pl.ds(0, _GA)], sem).wait()
        pltpu.make_async_copy(table_hbm.at[idx_v.at[c, pl.ds(_GA, _GB)]],
                              buf.at[pl.ds(_GA, _GB)], sem).wait()

    def fire_write(c, buf, sem):
        pltpu.async_copy(buf, out_hbm.at[b0 + c, :, pl.ds(0, _D)], sem)

    def wait_write(c, buf, sem):
        pltpu.make_async_copy(buf, out_hbm.at[b0 + c, :, pl.ds(0, _D)],
                              sem).wait()

    bufs = (rows0_v, rows1_v, rows2_v, rows3_v,
            rows4_v, rows5_v, rows6_v, rows7_v)
    sgs = (sg0, sg1, sg2, sg3, sg4, sg5, sg6, sg7)
    sws = (sw0, sw1, sw2, sw3, sw4, sw5, sw6, sw7)
    for j in range(8):
        fire_gather(j, bufs[j], sgs[j])

    def step(i, carry):
        c0 = i * 8
        for j in range(8):
            wait_gather(c0 + j, bufs[j], sgs[j])
            fire_write(c0 + j, bufs[j], sws[j])

        @pl.when(i < _NB // 8 - 1)
        def _refill():
            for j in range(8):
                wait_write(c0 + j, bufs[j], sws[j])
                fire_gather(c0 + 8 + j, bufs[j], sgs[j])

        return carry

    lax.fori_loop(0, _NB // 8, step, 0)
    for j in range(8):
        wait_write(_NB - 8 + j, bufs[j], sws[j])


_mesh = plsc.VectorSubcoreMesh(core_axis_name="c", subcore_axis_name="s")

_gather = functools.partial(
    pl.kernel,
    out_type=jax.ShapeDtypeStruct((_B, _S, _DP), jnp.float32),
    mesh=_mesh,
    scratch_types=(
        [pltpu.VMEM((_NB, _S), jnp.int32)]
        + [pltpu.VMEM((_S, _D), jnp.float32)] * 8
        + [pltpu.SemaphoreType.DMA] * 16
    ),
    compiler_params=pltpu.CompilerParams(use_tc_tiling_on_sc=False),
)(_gather_body)


def kernel(gene_indices, table):
    wide = _gather(gene_indices, table)
    return lax.slice(wide, (0, 0, 0), (_B, _S, _D))


# speedup vs baseline: 1.0044x; 1.0044x over previous
import functools

import jax
import jax.numpy as jnp
from jax import lax
from jax.experimental import pallas as pl
from jax.experimental.pallas import tpu as pltpu
from jax.experimental.pallas import tpu_sc as plsc

_B = 4096
_S = 200
_D = 64
_DP = 128
_NC = 2
_NS = 16
_NW = _NC * _NS
_NB = _B // _NW
_GA = 96
_GB = _S - _GA


def _gather_body(idx_hbm, table_hbm, out_hbm, idx_v,
                 rows0_v, rows1_v, rows2_v, rows3_v,
                 sg0, sg1, sg2, sg3, sw0, sw1, sw2, sw3):
    wid = lax.axis_index("s") * _NC + lax.axis_index("c")
    b0 = wid * _NB
    pltpu.sync_copy(idx_hbm.at[pl.ds(b0, _NB)], idx_v)

    def fire_gather(c, buf, sem):
        pltpu.async_copy(table_hbm.at[idx_v.at[c, pl.ds(0, _GA)]],
                         buf.at[pl.ds(0, _GA)], sem)
        pltpu.async_copy(table_hbm.at[idx_v.at[c, pl.ds(_GA, _GB)]],
                         buf.at[pl.ds(_GA, _GB)], sem)

    def wait_gather(c, buf, sem):
        pltpu.make_async_copy(table_hbm.at[idx_v.at[c, pl.ds(0, _GA)]],
                              buf.at[pl.ds(0, _GA)], sem).wait()
        pltpu.make_async_copy(table_hbm.at[idx_v.at[c, pl.ds(_GA, _GB)]],
                              buf.at[pl.ds(_GA, _GB)], sem).wait()

    def fire_write(c, buf, sem):
        pltpu.async_copy(buf, out_hbm.at[b0 + c, :, pl.ds(0, _D)], sem)

    def wait_write(c, buf, sem):
        pltpu.make_async_copy(buf, out_hbm.at[b0 + c, :, pl.ds(0, _D)],
                              sem).wait()

    bufs = (rows0_v, rows1_v, rows2_v, rows3_v)
    sgs = (sg0, sg1, sg2, sg3)
    sws = (sw0, sw1, sw2, sw3)
    for j in range(4):
        fire_gather(j, bufs[j], sgs[j])

    def step(i, carry):
        c0 = i * 4
        for j in range(4):
            wait_gather(c0 + j, bufs[j], sgs[j])
            fire_write(c0 + j, bufs[j], sws[j])

        @pl.when(i < _NB // 4 - 1)
        def _refill():
            for j in range(4):
                wait_write(c0 + j, bufs[j], sws[j])
                fire_gather(c0 + 4 + j, bufs[j], sgs[j])

        return carry

    lax.fori_loop(0, _NB // 4, step, 0)
    for j in range(4):
        wait_write(_NB - 4 + j, bufs[j], sws[j])


_mesh = plsc.VectorSubcoreMesh(core_axis_name="c", subcore_axis_name="s")

_gather = functools.partial(
    pl.kernel,
    out_type=jax.ShapeDtypeStruct((_B, _S, _DP), jnp.float32),
    mesh=_mesh,
    scratch_types=(
        [pltpu.VMEM((_NB, _S), jnp.int32)]
        + [pltpu.VMEM((_S, _D), jnp.float32)] * 4
        + [pltpu.SemaphoreType.DMA] * 8
    ),
    compiler_params=pltpu.CompilerParams(use_tc_tiling_on_sc=False),
)(_gather_body)


def kernel(gene_indices, table):
    wide = _gather(gene_indices, table)
    return lax.slice(wide, (0, 0, 0), (_B, _S, _D))
